# baseline (speedup 1.0000x reference)
import jax
import jax.numpy as jnp
from jax import lax
from jax.experimental import pallas as pl
from jax.experimental.pallas import tpu as pltpu

B, V = 16384, 1000


def _body(cols_ref, a_hbm, b_hbm, g_hbm, o_ref,
          a_v, b_v, g_v, sem_a, sem_b, sem_g):
    ia = cols_ref[0]
    ib = cols_ref[1]
    ig = cols_ref[2]
    ra = pl.multiple_of((ia // 8) * 8, 8)
    rb = pl.multiple_of((ib // 8) * 8, 8)
    rg = pl.multiple_of((ig // 8) * 8, 8)
    cp_a = pltpu.make_async_copy(a_hbm.at[pl.ds(ra, 8)], a_v, sem_a)
    cp_b = pltpu.make_async_copy(b_hbm.at[pl.ds(rb, 8)], b_v, sem_b)
    cp_g = pltpu.make_async_copy(g_hbm.at[pl.ds(rg, 8)], g_v, sem_g)
    cp_a.start()
    cp_b.start()
    cp_g.start()
    cp_a.wait()
    cp_b.wait()
    cp_g.wait()
    sub = lax.broadcasted_iota(jnp.int32, (8, B), 0)
    av = jnp.sum(jnp.where(sub == ia % 8, a_v[...], 0.0), axis=0)
    bv = jnp.sum(jnp.where(sub == ib % 8, b_v[...], 0.0), axis=0)
    gv = jnp.sum(jnp.where(sub == ig % 8, g_v[...], 0.0), axis=0)
    o_ref[...] = jnp.maximum(av + bv - gv, 0.0)


_call = pl.pallas_call(
    _body,
    in_specs=[
        pl.BlockSpec(memory_space=pltpu.MemorySpace.SMEM),
        pl.BlockSpec(memory_space=pltpu.MemorySpace.HBM),
        pl.BlockSpec(memory_space=pltpu.MemorySpace.HBM),
        pl.BlockSpec(memory_space=pltpu.MemorySpace.HBM),
    ],
    out_specs=pl.BlockSpec(memory_space=pltpu.MemorySpace.VMEM),
    out_shape=jax.ShapeDtypeStruct((B,), jnp.float32),
    scratch_shapes=[
        pltpu.VMEM((8, B), jnp.float32),
        pltpu.VMEM((8, B), jnp.float32),
        pltpu.VMEM((8, B), jnp.float32),
        pltpu.SemaphoreType.DMA,
        pltpu.SemaphoreType.DMA,
        pltpu.SemaphoreType.DMA,
    ],
)


def kernel(log_y_alpha, log_y_beta, log_y_gamma, alpha_index, beta_index, gamma_index):
    cols = jnp.stack([
        jnp.asarray(alpha_index, dtype=jnp.int32),
        jnp.asarray(beta_index, dtype=jnp.int32),
        jnp.asarray(gamma_index, dtype=jnp.int32),
    ])
    return _call(cols, log_y_alpha.T, log_y_beta.T, log_y_gamma.T)
